# Initial kernel scaffold; baseline (speedup 1.0000x reference)
import jax
import jax.numpy as jnp
from jax.experimental import pallas as pl


def kernel(x_user, x_item, block0_clicks, block0_clicked_by, block1_clicks, block1_clicked_by, pos_clicks, pos_clicked_by, neg_clicks, neg_clicked_by, W1_clicks, W1_cb, b1, W2_clicks, W2_cb, b2, Wp, bp):
    raise NotImplementedError("write your pallas kernel here")


# speedup vs baseline: 5.6116x; 5.6116x over previous
import functools

import jax
import jax.numpy as jnp
from jax import lax
from jax.experimental import pallas as pl
from jax.experimental.pallas import tpu as pltpu
from jax.experimental.pallas import tpu_sc as plsc

NC = 2
NS = 16
K = 128


def _mesh():
    return plsc.VectorSubcoreMesh(
        core_axis_name="c", subcore_axis_name="s", num_cores=NC,
        num_subcores=NS)


def _make_sc_segsum_wide(NP, PE):
    RPT = NP // NS
    RC = RPT // K
    CPT = PE // (NS * K)

    @functools.partial(
        pl.kernel,
        mesh=_mesh(),
        compiler_params=pltpu.CompilerParams(
            use_tc_tiling_on_sc=False, needs_layout_passes=False),
        out_type=[
            jax.ShapeDtypeStruct((NP, 128), jnp.float32),
            jax.ShapeDtypeStruct((NP, 128), jnp.float32),
            jax.ShapeDtypeStruct((NP, 16), jnp.float32),
            jax.ShapeDtypeStruct((NP, 128), jnp.float32),
            jax.ShapeDtypeStruct((NP, 128), jnp.float32),
            jax.ShapeDtypeStruct((NP, 16), jnp.float32),
        ],
        scratch_types=[
            pltpu.VMEM((K,), jnp.int32),
            pltpu.VMEM((K,), jnp.int32),
            pltpu.VMEM((K, 128), jnp.float32),
            pltpu.VMEM((K, 16), jnp.float32),
            pltpu.VMEM_SHARED((NP, 128), jnp.float32),
            pltpu.VMEM_SHARED((NP, 16), jnp.float32),
            pltpu.SemaphoreType.DMA,
        ],
    )
    def sc_segsum_wide(xa_lo, xa_hi, esa, eda, xb_lo, xb_hi, esb, edb,
                       z128, z16, ones16,
                       agg_a_lo, agg_a_hi, deg_a, agg_b_lo, agg_b_hi, deg_b,
                       idxs_v, idxd_v, rows_v, ones_v,
                       acc_sh, deg_sh, sem):
        c = lax.axis_index("c")
        s = lax.axis_index("s")
        rbase = s * RPT

        def zero_acc():
            pltpu.sync_copy(z128, rows_v)
            for j in range(RC):
                pltpu.sync_copy(rows_v, acc_sh.at[pl.ds(rbase + j * K, K)])

            @pl.when(c == 0)
            def _():
                pltpu.sync_copy(z16, ones_v)
                for j in range(RC):
                    pltpu.sync_copy(
                        ones_v, deg_sh.at[pl.ds(rbase + j * K, K)])
                pltpu.sync_copy(ones16, ones_v)

        def do_edges(x_lo, x_hi, es, ed):
            def loop(src_hbm, do_deg):
                def body(i, carry):
                    cb = (s * CPT + i) * K
                    pltpu.sync_copy(es.at[pl.ds(cb, K)], idxs_v)
                    pltpu.sync_copy(ed.at[pl.ds(cb, K)], idxd_v)
                    pltpu.async_copy(src_hbm.at[idxs_v], rows_v, sem).wait()
                    pltpu.sync_copy(rows_v, acc_sh.at[idxd_v], add=True)
                    if do_deg:
                        pltpu.sync_copy(ones_v, deg_sh.at[idxd_v], add=True)
                    return carry
                lax.fori_loop(0, CPT, body, 0)

            @pl.when(c == 0)
            def _():
                loop(x_lo, True)

            @pl.when(c == 1)
            def _():
                loop(x_hi, False)

        def writeback(agg_lo, agg_hi, deg16):
            def wb_rows(agg_out):
                for j in range(RC):
                    pltpu.sync_copy(acc_sh.at[pl.ds(rbase + j * K, K)],
                                    rows_v)
                    pltpu.sync_copy(rows_v, agg_out.at[pl.ds(rbase + j * K,
                                                             K)])

            @pl.when(c == 0)
            def _():
                wb_rows(agg_lo)
                for j in range(RC):
                    pltpu.sync_copy(deg_sh.at[pl.ds(rbase + j * K, K)],
                                    ones_v)
                    pltpu.sync_copy(ones_v, deg16.at[pl.ds(rbase + j * K,
                                                           K)])

            @pl.when(c == 1)
            def _():
                wb_rows(agg_hi)

        zero_acc()
        plsc.subcore_barrier()
        do_edges(xa_lo, xa_hi, esa, eda)
        plsc.subcore_barrier()
        writeback(agg_a_lo, agg_a_hi, deg_a)
        zero_acc()
        plsc.subcore_barrier()
        do_edges(xb_lo, xb_hi, esb, edb)
        plsc.subcore_barrier()
        writeback(agg_b_lo, agg_b_hi, deg_b)

    return sc_segsum_wide


def _make_sc_segsum_narrow(NP, PE):
    RPT = NP // NS
    RC = RPT // K
    CPT = PE // (NS * K)

    @functools.partial(
        pl.kernel,
        mesh=_mesh(),
        compiler_params=pltpu.CompilerParams(
            use_tc_tiling_on_sc=False, needs_layout_passes=False),
        out_type=[
            jax.ShapeDtypeStruct((NP, 16), jnp.float32),
            jax.ShapeDtypeStruct((NP, 16), jnp.float32),
        ],
        scratch_types=[
            pltpu.VMEM((K,), jnp.int32),
            pltpu.VMEM((K,), jnp.int32),
            pltpu.VMEM((K, 16), jnp.float32),
            pltpu.VMEM((K, 16), jnp.float32),
            pltpu.VMEM_SHARED((NP, 16), jnp.float32),
            pltpu.SemaphoreType.DMA,
        ],
    )
    def sc_segsum_narrow(c16u, c16i, ec0, ec1, eb0, eb1, z16,
                         acc_item, acc_user,
                         idxs_v, idxd_v, rows_v, stag_v, acc_sh, sem):
        c = lax.axis_index("c")
        s = lax.axis_index("s")
        rbase = s * RPT

        pltpu.sync_copy(z16, stag_v)
        for j in range(RC):
            pltpu.sync_copy(stag_v, acc_sh.at[pl.ds(rbase + j * K, K)])
        plsc.subcore_barrier()

        def do_edges(src_hbm, e0, e1):
            def body(i, carry):
                cb = (s * CPT + i) * K
                pltpu.sync_copy(e0.at[pl.ds(cb, K)], idxs_v)
                pltpu.sync_copy(e1.at[pl.ds(cb, K)], idxd_v)
                pltpu.async_copy(src_hbm.at[idxs_v], rows_v, sem).wait()
                pltpu.sync_copy(rows_v, acc_sh.at[idxd_v], add=True)
                return carry
            lax.fori_loop(0, CPT, body, 0)

        @pl.when(c == 0)
        def _():
            do_edges(c16u, ec0, ec1)

        @pl.when(c == 1)
        def _():
            do_edges(c16i, eb0, eb1)

        plsc.subcore_barrier()

        def wb(out):
            for j in range(RC):
                pltpu.sync_copy(acc_sh.at[pl.ds(rbase + j * K, K)], stag_v)
                pltpu.sync_copy(stag_v, out.at[pl.ds(rbase + j * K, K)])

        @pl.when(c == 0)
        def _():
            wb(acc_item)

        @pl.when(c == 1)
        def _():
            wb(acc_user)

    return sc_segsum_narrow


def _make_sc_score(NP, PE):
    EPT = PE // (NC * NS)

    @functools.partial(
        pl.kernel,
        mesh=_mesh(),
        compiler_params=pltpu.CompilerParams(
            use_tc_tiling_on_sc=False, needs_layout_passes=False),
        out_type=[jax.ShapeDtypeStruct((PE,), jnp.float32)
                  for _ in range(4)],
        scratch_types=[
            pltpu.VMEM((NP,), jnp.float32),
            pltpu.VMEM((NP,), jnp.float32),
            pltpu.VMEM((NP,), jnp.float32),
            pltpu.VMEM((NP,), jnp.float32),
            pltpu.VMEM((EPT,), jnp.int32),
            pltpu.VMEM((EPT,), jnp.int32),
            pltpu.VMEM((EPT,), jnp.float32),
        ],
    )
    def sc_score(au, bu, ai, bi, pc0, pc1, pb0, pb1, nn0, nn1, nb0, nb1,
                 o_pc, o_pb, o_nc, o_nb,
                 au_v, bu_v, ai_v, bi_v, e0_v, e1_v, out_v):
        c = lax.axis_index("c")
        s = lax.axis_index("s")
        wid = s * NC + c
        base = wid * EPT
        pltpu.sync_copy(au, au_v)
        pltpu.sync_copy(bu, bu_v)
        pltpu.sync_copy(ai, ai_v)
        pltpu.sync_copy(bi, bi_v)
        lists = ((pc0, pc1, o_pc, True), (pb0, pb1, o_pb, False),
                 (nn0, nn1, o_nc, True), (nb0, nb1, o_nb, False))
        for e0, e1, out, clicks in lists:
            pltpu.sync_copy(e0.at[pl.ds(base, EPT)], e0_v)
            pltpu.sync_copy(e1.at[pl.ds(base, EPT)], e1_v)
            ref_a, ref_b = (au_v, bi_v) if clicks else (ai_v, bu_v)

            def body(j, carry, ref_a=ref_a, ref_b=ref_b):
                i0 = e0_v[pl.ds(j * 16, 16)]
                i1 = e1_v[pl.ds(j * 16, 16)]
                a = plsc.load_gather(ref_a, [i0])
                b = plsc.load_gather(ref_b, [i1])
                out_v[pl.ds(j * 16, 16)] = a + b
                return carry
            lax.fori_loop(0, EPT // 16, body, 0)
            pltpu.sync_copy(out_v, out.at[pl.ds(base, EPT)])

    return sc_score


def _make_tc_project(NP, RB):
    grid = NP // RB

    def body(agg_lo, agg_hi, deg, w1, b1, w2, wpc, out):
        d = jnp.maximum(deg[:, 0:1], 1.0)
        lo = agg_lo[...] / d
        hi = agg_hi[...] / d
        h = jnp.dot(lo, w1[0:128, :], preferred_element_type=jnp.float32,
                    precision=lax.Precision.HIGHEST)
        h = h + jnp.dot(hi, w1[128:256, :],
                        preferred_element_type=jnp.float32,
                    precision=lax.Precision.HIGHEST)
        h = jnp.maximum(h + b1[...], 0.0)
        u = jnp.dot(w2[...], wpc[...], preferred_element_type=jnp.float32,
                    precision=lax.Precision.HIGHEST)
        c2 = jnp.dot(h, u, preferred_element_type=jnp.float32,
                    precision=lax.Precision.HIGHEST)
        out[:, 0:2] = c2
        out[:, 2:3] = jnp.ones_like(d)
        out[:, 3:16] = jnp.zeros((RB, 13), jnp.float32)

    return pl.pallas_call(
        body,
        grid=(grid,),
        in_specs=[
            pl.BlockSpec((RB, 128), lambda i: (i, 0)),
            pl.BlockSpec((RB, 128), lambda i: (i, 0)),
            pl.BlockSpec((RB, 16), lambda i: (i, 0)),
            pl.BlockSpec((256, 256), lambda i: (0, 0)),
            pl.BlockSpec((1, 256), lambda i: (0, 0)),
            pl.BlockSpec((256, 128), lambda i: (0, 0)),
            pl.BlockSpec((128, 2), lambda i: (0, 0)),
        ],
        out_specs=pl.BlockSpec((RB, 16), lambda i: (i, 0)),
        out_shape=jax.ShapeDtypeStruct((NP, 16), jnp.float32),
    )


def _make_tc_finalize(NP, RB):
    grid = NP // RB

    def body(accu, acci, b2, wpc, bp, outu, outi):
        cc = jnp.dot(b2[...], wpc[...], preferred_element_type=jnp.float32,
                    precision=lax.Precision.HIGHEST)
        for acc, out in ((accu, outu), (acci, outi)):
            a = acc[:, 0:1]
            b = acc[:, 1:2]
            d = jnp.maximum(acc[:, 2:3], 1.0)
            out[:, 0:1] = a / d + cc[0:1, 0:1] + bp[...]
            out[:, 1:2] = b / d + cc[0:1, 1:2]

    return pl.pallas_call(
        body,
        grid=(grid,),
        in_specs=[
            pl.BlockSpec((RB, 16), lambda i: (i, 0)),
            pl.BlockSpec((RB, 16), lambda i: (i, 0)),
            pl.BlockSpec((1, 128), lambda i: (0, 0)),
            pl.BlockSpec((128, 2), lambda i: (0, 0)),
            pl.BlockSpec((1, 1), lambda i: (0, 0)),
        ],
        out_specs=[
            pl.BlockSpec((RB, 2), lambda i: (i, 0)),
            pl.BlockSpec((RB, 2), lambda i: (i, 0)),
        ],
        out_shape=[
            jax.ShapeDtypeStruct((NP, 2), jnp.float32),
            jax.ShapeDtypeStruct((NP, 2), jnp.float32),
        ],
    )


def kernel(x_user, x_item,
           block0_clicks, block0_clicked_by,
           block1_clicks, block1_clicked_by,
           pos_clicks, pos_clicked_by,
           neg_clicks, neg_clicked_by,
           W1_clicks, W1_cb, b1,
           W2_clicks, W2_cb, b2,
           Wp, bp):
    N = x_user.shape[0]
    E = block0_clicks.shape[1]
    CH = NS * K
    NP = -(-(N + 1) // CH) * CH
    PE = -(-E // CH) * CH
    RB = NP // 4

    def pad_edges(e):
        e = e.astype(jnp.int32)
        fill = jnp.full((PE - E,), N, jnp.int32)
        return (jnp.concatenate([e[0], fill]),
                jnp.concatenate([e[1], fill]))

    xu = jnp.pad(x_user, ((0, NP - N), (0, 0)))
    xi = jnp.pad(x_item, ((0, NP - N), (0, 0)))
    z128 = jnp.zeros((K, 128), jnp.float32)
    z16 = jnp.zeros((K, 16), jnp.float32)
    ones16 = jnp.ones((K, 16), jnp.float32)

    b1r = b1.reshape(1, 256)
    b2r = b2.reshape(1, 128)
    bpr = bp.reshape(1, 1)
    wpc = jnp.concatenate([Wp[:128, 0:1], Wp[128:, 0:1]], axis=1)

    segsum1 = _make_sc_segsum_wide(NP, PE)
    segsum2 = _make_sc_segsum_narrow(NP, PE)
    score = _make_sc_score(NP, PE)
    project = _make_tc_project(NP, RB)
    finalize = _make_tc_finalize(NP, RB)

    e0a, e1a = pad_edges(block0_clicks)
    e0b, e1b = pad_edges(block0_clicked_by)
    (agg_i_lo, agg_i_hi, deg_i, agg_u_lo, agg_u_hi, deg_u) = segsum1(
        xu[:, :128], xu[:, 128:], e0a, e1a,
        xi[:, :128], xi[:, 128:], e0b, e1b, z128, z16, ones16)

    c16_item = project(agg_i_lo, agg_i_hi, deg_i, W1_clicks, b1r, W2_cb, wpc)
    c16_user = project(agg_u_lo, agg_u_hi, deg_u, W1_cb, b1r, W2_clicks, wpc)

    ec0, ec1 = pad_edges(block1_clicks)
    eb0, eb1 = pad_edges(block1_clicked_by)
    acc_item, acc_user = segsum2(c16_user, c16_item, ec0, ec1, eb0, eb1, z16)

    user2, item2 = finalize(acc_user, acc_item, b2r, wpc, bpr)

    pc0, pc1 = pad_edges(pos_clicks)
    pb0, pb1 = pad_edges(pos_clicked_by)
    nn0, nn1 = pad_edges(neg_clicks)
    nb0, nb1 = pad_edges(neg_clicked_by)
    s_pc, s_pb, s_nc, s_nb = score(
        user2[:, 0], user2[:, 1], item2[:, 0], item2[:, 1],
        pc0, pc1, pb0, pb1, nn0, nn1, nb0, nb1)

    pos_score = jnp.concatenate([s_pc[:E], s_pb[:E]])
    neg_score = jnp.concatenate([s_nc[:E], s_nb[:E]])
    return (pos_score, neg_score)
